# Initial kernel scaffold; baseline (speedup 1.0000x reference)
import jax
import jax.numpy as jnp
from jax.experimental import pallas as pl


def kernel(x, noise):
    raise NotImplementedError("write your pallas kernel here")


# speedup vs baseline: 1.2943x; 1.2943x over previous
import functools

import jax
import jax.numpy as jnp
from jax import lax
from jax.experimental import pallas as pl
from jax.experimental.pallas import tpu as pltpu
from jax.experimental.pallas import tpu_sc as plsc

B, N, D = 64, 576, 768
K = 288
BB = 8

NW = 32
ROWS_PER_W = (B * K) // NW
CHUNK = 64
NCHUNKS = ROWS_PER_W // CHUNK


def _total_order_key(v):
    bits = lax.bitcast_convert_type(v, jnp.int32)
    return bits ^ ((bits >> 31) & jnp.int32(0x7FFFFFFF))


def _rank_body(noise_ref, noise_t_ref, idx_ref):
    kb = _total_order_key(noise_ref[...])
    kt = _total_order_key(noise_t_ref[0])
    ii = lax.broadcasted_iota(jnp.int32, (N, 1), 0)
    jj = lax.broadcasted_iota(jnp.int32, (1, N), 1)
    rr = lax.broadcasted_iota(jnp.int32, (1, K), 1)
    for b in range(BB):
        ki = kt[:, b:b + 1]
        kj = kb[b:b + 1, :]
        beats = (kj > ki) | ((kj == ki) & (jj < ii))
        rank = jnp.sum(beats.astype(jnp.int32), axis=1, keepdims=True)
        sel = (rank == rr).astype(jnp.int32)
        idxv = jnp.sum(sel * ii, axis=0, keepdims=True)
        row = pl.program_id(0) * BB + b
        idx_ref[b:b + 1, :] = idxv + row * N


def _topk_flat_indices(noise):
    noise_t = noise.reshape(B // BB, BB, N).transpose(0, 2, 1)
    return pl.pallas_call(
        _rank_body,
        grid=(B // BB,),
        in_specs=[
            pl.BlockSpec((BB, N), lambda i: (i, 0)),
            pl.BlockSpec((1, N, BB), lambda i: (i, 0, 0)),
        ],
        out_specs=pl.BlockSpec((BB, K), lambda i: (i, 0)),
        out_shape=jax.ShapeDtypeStruct((B, K), jnp.int32),
    )(noise, noise_t)


def _sc_gather(x_flat, idx_flat):
    mesh = plsc.VectorSubcoreMesh(core_axis_name="c", subcore_axis_name="s")

    @functools.partial(
        pl.kernel,
        mesh=mesh,
        out_type=jax.ShapeDtypeStruct((B * K, D), jnp.float32),
        scratch_types=[
            pltpu.VMEM((ROWS_PER_W,), jnp.int32),
            pltpu.VMEM((CHUNK, D), jnp.float32),
            pltpu.VMEM((CHUNK, D), jnp.float32),
            pltpu.SemaphoreType.DMA,
            pltpu.SemaphoreType.DMA,
        ],
    )
    def gather_kernel(x_hbm, idx_hbm, out_hbm, idx_v, buf0, buf1, sem0, sem1):
        wid = lax.axis_index("s") * 2 + lax.axis_index("c")
        base = wid * ROWS_PER_W
        pltpu.sync_copy(idx_hbm.at[pl.ds(base, ROWS_PER_W)], idx_v)
        bufs = (buf0, buf1)
        sems = (sem0, sem1)
        copies = [None] * NCHUNKS
        copies[0] = pltpu.async_copy(
            x_hbm.at[idx_v.at[pl.ds(0, CHUNK)]], bufs[0], sems[0])
        for c in range(NCHUNKS):
            if c + 1 < NCHUNKS:
                copies[c + 1] = pltpu.async_copy(
                    x_hbm.at[idx_v.at[pl.ds((c + 1) * CHUNK, CHUNK)]],
                    bufs[(c + 1) % 2], sems[(c + 1) % 2])
            copies[c].wait()
            pltpu.sync_copy(bufs[c % 2],
                            out_hbm.at[pl.ds(base + c * CHUNK, CHUNK)])

    return gather_kernel(x_flat, idx_flat)


@jax.jit
def kernel(x, noise):
    idx = _topk_flat_indices(noise)
    out_flat = _sc_gather(x.reshape(B * N, D), idx.reshape(B * K))
    return out_flat.reshape(B, K, D)
